# baseline (speedup 1.0000x reference)
import functools
import jax
import jax.numpy as jnp
from jax import lax
from jax.experimental import pallas as pl
from jax.experimental.pallas import tpu as pltpu
from jax.experimental.pallas import tpu_sc as plsc

B = 16384
D = 64
NC = 2
NS = 16
NW = NC * NS
RPW = B // NW
CHUNK = 128
NCHUNK = RPW // CHUNK


def _gather_body(ut_hbm, mt_hbm, uidx_hbm, midx_hbm, ue_hbm, me_hbm,
                 uidx_v, midx_v, urows_v, mrows_v, sem):
    wid = lax.axis_index("s") * NC + lax.axis_index("c")
    cbase = wid * NCHUNK
    pltpu.sync_copy(uidx_hbm.at[pl.ds(cbase, NCHUNK)], uidx_v)
    pltpu.sync_copy(midx_hbm.at[pl.ds(cbase, NCHUNK)], midx_v)
    copies = []
    for j in range(NCHUNK):
        copies.append(pltpu.async_copy(
            ut_hbm.at[uidx_v.at[j]], urows_v.at[pl.ds(j * CHUNK, CHUNK)], sem))
        copies.append(pltpu.async_copy(
            mt_hbm.at[midx_v.at[j]], mrows_v.at[pl.ds(j * CHUNK, CHUNK)], sem))
    for c in copies:
        c.wait()
    rbase = wid * RPW
    pltpu.sync_copy(urows_v, ue_hbm.at[pl.ds(rbase, RPW)])
    pltpu.sync_copy(mrows_v, me_hbm.at[pl.ds(rbase, RPW)])


@functools.lru_cache(maxsize=1)
def _make_gather_call():
    return functools.partial(
        pl.kernel,
        out_type=(
            jax.ShapeDtypeStruct((B, D), jnp.float32),
            jax.ShapeDtypeStruct((B, D), jnp.float32),
        ),
        mesh=plsc.VectorSubcoreMesh(core_axis_name="c", subcore_axis_name="s"),
        compiler_params=pltpu.CompilerParams(use_tc_tiling_on_sc=False),
        scratch_types=[
            pltpu.VMEM((NCHUNK, CHUNK), jnp.int32),
            pltpu.VMEM((NCHUNK, CHUNK), jnp.int32),
            pltpu.VMEM((RPW, D), jnp.float32),
            pltpu.VMEM((RPW, D), jnp.float32),
            pltpu.SemaphoreType.DMA,
        ],
    )(_gather_body)


BLK = 2048


def _dense_body(ue, me, uf, mf, uW1, ub1, uW2, ub2, mW1, mb1, mW2, mb2,
                dW1, db1, dW2, db2, dW3, db3, out):
    f32 = jnp.float32
    ufe = jnp.maximum(jnp.dot(uf[...], uW1[...], preferred_element_type=f32)
                      + ub1[...], 0.0)
    ufe = jnp.dot(ufe, uW2[...], preferred_element_type=f32) + ub2[...]
    mfe = jnp.maximum(jnp.dot(mf[...], mW1[...], preferred_element_type=f32)
                      + mb1[...], 0.0)
    mfe = jnp.dot(mfe, mW2[...], preferred_element_type=f32) + mb2[...]
    h = (jnp.dot(ue[...], dW1[0:64, :], preferred_element_type=f32)
         + jnp.dot(me[...], dW1[64:128, :], preferred_element_type=f32)
         + jnp.dot(ufe, dW1[128:160, :], preferred_element_type=f32)
         + jnp.dot(mfe, dW1[160:192, :], preferred_element_type=f32)
         + db1[...])
    h = jnp.maximum(h, 0.0)
    h = jnp.maximum(jnp.dot(h, dW2[...], preferred_element_type=f32)
                    + db2[...], 0.0)
    out[...] = jnp.dot(h, dW3[...], preferred_element_type=f32) + db3[...]


def _dense_call(ue, me, uf, mf, uW1, ub1, uW2, ub2, mW1, mb1, mW2, mb2,
                dW1, db1, dW2, db2, dW3, db3, interpret=False):
    row_spec = pl.BlockSpec((BLK, D), lambda i: (i, 0))
    full = lambda a: pl.BlockSpec(a.shape, lambda i: tuple(0 for _ in a.shape))
    args = (ue, me, uf, mf, uW1, ub1, uW2, ub2, mW1, mb1, mW2, mb2,
            dW1, db1, dW2, db2, dW3, db3)
    in_specs = [row_spec, row_spec, row_spec, row_spec] + [full(a) for a in args[4:]]
    return pl.pallas_call(
        _dense_body,
        grid=(B // BLK,),
        in_specs=in_specs,
        out_specs=pl.BlockSpec((BLK, 1), lambda i: (i, 0)),
        out_shape=jax.ShapeDtypeStruct((B, 1), jnp.float32),
        interpret=interpret,
    )(*args)


def kernel(user_ids, movie_ids, user_features, movie_features, user_table,
           movie_table, uW1, ub1, uW2, ub2, mW1, mb1, mW2, mb2,
           dW1, db1, dW2, db2, dW3, db3):
    uidx = user_ids.astype(jnp.int32).reshape(NW * NCHUNK, CHUNK)
    midx = movie_ids.astype(jnp.int32).reshape(NW * NCHUNK, CHUNK)
    ue, me = _make_gather_call()(user_table, movie_table, uidx, midx)
    out = _dense_call(
        ue, me, user_features, movie_features,
        uW1, ub1.reshape(1, -1), uW2, ub2.reshape(1, -1),
        mW1, mb1.reshape(1, -1), mW2, mb2.reshape(1, -1),
        dW1, db1.reshape(1, -1), dW2, db2.reshape(1, -1),
        dW3, db3.reshape(1, -1))
    return out[:, 0]


# speedup vs baseline: 1.5395x; 1.5395x over previous
import functools
import jax
import jax.numpy as jnp
from jax import lax
from jax.experimental import pallas as pl
from jax.experimental.pallas import tpu as pltpu
from jax.experimental.pallas import tpu_sc as plsc

B = 16384
D = 64
EU = 1000000
EM = 100000
NC = 2
NS = 16
NW = NC * NS
RPW = B // NW
L = 16
NCHUNK = RPW // L


def _gather_table(tbl3, ids_v, rows_g, stage_v, out3, wid, sem):
    iota = lax.iota(jnp.int32, L)
    rb = iota >> 3
    rs = iota & 7

    def chunk(i, _):
        base = i * L
        idv = ids_v[pl.ds(base, L)]
        tid = idv >> 3
        sub = idv & 7
        copies = [
            pltpu.async_copy(tbl3.at[tid[j]], rows_g.at[j], sem)
            for j in range(L)
        ]
        for c in copies:
            c.wait()
        for c in range(D):
            col = jnp.full((L,), c, jnp.int32)
            v = plsc.load_gather(rows_g, [iota, sub, col])
            plsc.store_scatter(stage_v, [rb, rs, col], v)
        pltpu.sync_copy(stage_v, out3.at[pl.ds(wid * (RPW // 8) + i * 2, 2)])
        return ()

    lax.fori_loop(0, NCHUNK, chunk, (), unroll=False)


def _gather_body(ut3, mt3, uids_hbm, mids_hbm, ue3, me3,
                 uids_v, mids_v, rows_g, stage_v, sem):
    wid = lax.axis_index("s") * NC + lax.axis_index("c")
    rbase = wid * RPW
    pltpu.sync_copy(uids_hbm.at[pl.ds(rbase, RPW)], uids_v)
    pltpu.sync_copy(mids_hbm.at[pl.ds(rbase, RPW)], mids_v)
    _gather_table(ut3, uids_v, rows_g, stage_v, ue3, wid, sem)
    _gather_table(mt3, mids_v, rows_g, stage_v, me3, wid, sem)


@functools.lru_cache(maxsize=1)
def _make_gather_call():
    return functools.partial(
        pl.kernel,
        out_type=(
            jax.ShapeDtypeStruct((B // 8, 8, D), jnp.float32),
            jax.ShapeDtypeStruct((B // 8, 8, D), jnp.float32),
        ),
        mesh=plsc.VectorSubcoreMesh(core_axis_name="c", subcore_axis_name="s"),
        compiler_params=pltpu.CompilerParams(needs_layout_passes=False),
        scratch_types=[
            pltpu.VMEM((RPW,), jnp.int32),
            pltpu.VMEM((RPW,), jnp.int32),
            pltpu.VMEM((L, 8, D), jnp.float32),
            pltpu.VMEM((2, 8, D), jnp.float32),
            pltpu.SemaphoreType.DMA,
        ],
    )(_gather_body)


BLK = 2048


def _dense_body(ue, me, uf, mf, uW1, ub1, uW2, ub2, mW1, mb1, mW2, mb2,
                dW1, db1, dW2, db2, dW3, db3, out):
    f32 = jnp.float32
    ufe = jnp.maximum(jnp.dot(uf[...], uW1[...], preferred_element_type=f32)
                      + ub1[...], 0.0)
    ufe = jnp.dot(ufe, uW2[...], preferred_element_type=f32) + ub2[...]
    mfe = jnp.maximum(jnp.dot(mf[...], mW1[...], preferred_element_type=f32)
                      + mb1[...], 0.0)
    mfe = jnp.dot(mfe, mW2[...], preferred_element_type=f32) + mb2[...]
    h = (jnp.dot(ue[...], dW1[0:64, :], preferred_element_type=f32)
         + jnp.dot(me[...], dW1[64:128, :], preferred_element_type=f32)
         + jnp.dot(ufe, dW1[128:160, :], preferred_element_type=f32)
         + jnp.dot(mfe, dW1[160:192, :], preferred_element_type=f32)
         + db1[...])
    h = jnp.maximum(h, 0.0)
    h = jnp.maximum(jnp.dot(h, dW2[...], preferred_element_type=f32)
                    + db2[...], 0.0)
    out[...] = jnp.dot(h, dW3[...], preferred_element_type=f32) + db3[...]


def _dense_call(ue, me, uf, mf, uW1, ub1, uW2, ub2, mW1, mb1, mW2, mb2,
                dW1, db1, dW2, db2, dW3, db3, interpret=False):
    row_spec = pl.BlockSpec((BLK, D), lambda i: (i, 0))
    full = lambda a: pl.BlockSpec(a.shape, lambda i: tuple(0 for _ in a.shape))
    args = (ue, me, uf, mf, uW1, ub1, uW2, ub2, mW1, mb1, mW2, mb2,
            dW1, db1, dW2, db2, dW3, db3)
    in_specs = [row_spec, row_spec, row_spec, row_spec] + [full(a) for a in args[4:]]
    return pl.pallas_call(
        _dense_body,
        grid=(B // BLK,),
        in_specs=in_specs,
        out_specs=pl.BlockSpec((BLK, 1), lambda i: (i, 0)),
        out_shape=jax.ShapeDtypeStruct((B, 1), jnp.float32),
        interpret=interpret,
    )(*args)


def kernel(user_ids, movie_ids, user_features, movie_features, user_table,
           movie_table, uW1, ub1, uW2, ub2, mW1, mb1, mW2, mb2,
           dW1, db1, dW2, db2, dW3, db3):
    ut3 = user_table.reshape(EU // 8, 8, D)
    mt3 = movie_table.reshape(EM // 8, 8, D)
    uids = user_ids.astype(jnp.int32)
    mids = movie_ids.astype(jnp.int32)
    ue3, me3 = _make_gather_call()(ut3, mt3, uids, mids)
    ue = ue3.reshape(B, D)
    me = me3.reshape(B, D)
    out = _dense_call(
        ue, me, user_features, movie_features,
        uW1, ub1.reshape(1, -1), uW2, ub2.reshape(1, -1),
        mW1, mb1.reshape(1, -1), mW2, mb2.reshape(1, -1),
        dW1, db1.reshape(1, -1), dW2, db2.reshape(1, -1),
        dW3, db3.reshape(1, -1))
    return out[:, 0]


# speedup vs baseline: 1.7707x; 1.1501x over previous
import functools
import jax
import jax.numpy as jnp
from jax import lax
from jax.experimental import pallas as pl
from jax.experimental.pallas import tpu as pltpu
from jax.experimental.pallas import tpu_sc as plsc

B = 16384
D = 64
EU = 1000000
EM = 100000
NC = 2
NS = 16
NW = NC * NS
RPW = B // NW
L = 16
NCHUNK = RPW // L


def _gather_table(tbl3, ids_v, rows_g, stage_v, out3, wid, sem):
    def chunk(i, _):
        base = i * L
        idv = ids_v[pl.ds(base, L)]
        tid = idv >> 3
        sub = idv & 7
        copies = [
            pltpu.async_copy(tbl3.at[tid[j]], rows_g.at[j], sem)
            for j in range(L)
        ]
        for c in copies:
            c.wait()
        for j in range(L):
            sj = sub[j]
            for k in range(0, D, L):
                stage_v[j >> 3, j & 7, pl.ds(k, L)] = rows_g[j, sj, pl.ds(k, L)]
        pltpu.sync_copy(stage_v, out3.at[pl.ds(wid * (RPW // 8) + i * 2, 2)])
        return ()

    lax.fori_loop(0, NCHUNK, chunk, (), unroll=False)


def _gather_body(ut3, mt3, uids_hbm, mids_hbm, ue3, me3,
                 uids_v, mids_v, rows_g, stage_v, sem):
    wid = lax.axis_index("s") * NC + lax.axis_index("c")
    rbase = wid * RPW
    pltpu.sync_copy(uids_hbm.at[pl.ds(rbase, RPW)], uids_v)
    pltpu.sync_copy(mids_hbm.at[pl.ds(rbase, RPW)], mids_v)
    _gather_table(ut3, uids_v, rows_g, stage_v, ue3, wid, sem)
    _gather_table(mt3, mids_v, rows_g, stage_v, me3, wid, sem)


@functools.lru_cache(maxsize=1)
def _make_gather_call():
    return functools.partial(
        pl.kernel,
        out_type=(
            jax.ShapeDtypeStruct((B // 8, 8, D), jnp.float32),
            jax.ShapeDtypeStruct((B // 8, 8, D), jnp.float32),
        ),
        mesh=plsc.VectorSubcoreMesh(core_axis_name="c", subcore_axis_name="s"),
        scratch_types=[
            pltpu.VMEM((RPW,), jnp.int32),
            pltpu.VMEM((RPW,), jnp.int32),
            pltpu.VMEM((L, 8, D), jnp.float32),
            pltpu.VMEM((2, 8, D), jnp.float32),
            pltpu.SemaphoreType.DMA,
        ],
    )(_gather_body)


BLK = 2048


def _dense_body(ue, me, uf, mf, uW1, ub1, uW2, ub2, mW1, mb1, mW2, mb2,
                dW1, db1, dW2, db2, dW3, db3, out):
    f32 = jnp.float32
    ufe = jnp.maximum(jnp.dot(uf[...], uW1[...], preferred_element_type=f32)
                      + ub1[...], 0.0)
    ufe = jnp.dot(ufe, uW2[...], preferred_element_type=f32) + ub2[...]
    mfe = jnp.maximum(jnp.dot(mf[...], mW1[...], preferred_element_type=f32)
                      + mb1[...], 0.0)
    mfe = jnp.dot(mfe, mW2[...], preferred_element_type=f32) + mb2[...]
    h = (jnp.dot(ue[...], dW1[0:64, :], preferred_element_type=f32)
         + jnp.dot(me[...], dW1[64:128, :], preferred_element_type=f32)
         + jnp.dot(ufe, dW1[128:160, :], preferred_element_type=f32)
         + jnp.dot(mfe, dW1[160:192, :], preferred_element_type=f32)
         + db1[...])
    h = jnp.maximum(h, 0.0)
    h = jnp.maximum(jnp.dot(h, dW2[...], preferred_element_type=f32)
                    + db2[...], 0.0)
    out[...] = jnp.dot(h, dW3[...], preferred_element_type=f32) + db3[...]


def _dense_call(ue, me, uf, mf, uW1, ub1, uW2, ub2, mW1, mb1, mW2, mb2,
                dW1, db1, dW2, db2, dW3, db3, interpret=False):
    row_spec = pl.BlockSpec((BLK, D), lambda i: (i, 0))
    full = lambda a: pl.BlockSpec(a.shape, lambda i: tuple(0 for _ in a.shape))
    args = (ue, me, uf, mf, uW1, ub1, uW2, ub2, mW1, mb1, mW2, mb2,
            dW1, db1, dW2, db2, dW3, db3)
    in_specs = [row_spec, row_spec, row_spec, row_spec] + [full(a) for a in args[4:]]
    return pl.pallas_call(
        _dense_body,
        grid=(B // BLK,),
        in_specs=in_specs,
        out_specs=pl.BlockSpec((BLK, 1), lambda i: (i, 0)),
        out_shape=jax.ShapeDtypeStruct((B, 1), jnp.float32),
        interpret=interpret,
    )(*args)


def kernel(user_ids, movie_ids, user_features, movie_features, user_table,
           movie_table, uW1, ub1, uW2, ub2, mW1, mb1, mW2, mb2,
           dW1, db1, dW2, db2, dW3, db3):
    ut3 = user_table.reshape(EU // 8, 8, D)
    mt3 = movie_table.reshape(EM // 8, 8, D)
    uids = user_ids.astype(jnp.int32)
    mids = movie_ids.astype(jnp.int32)
    ue3, me3 = _make_gather_call()(ut3, mt3, uids, mids)
    ue = ue3.reshape(B, D)
    me = me3.reshape(B, D)
    out = _dense_call(
        ue, me, user_features, movie_features,
        uW1, ub1.reshape(1, -1), uW2, ub2.reshape(1, -1),
        mW1, mb1.reshape(1, -1), mW2, mb2.reshape(1, -1),
        dW1, db1.reshape(1, -1), dW2, db2.reshape(1, -1),
        dW3, db3.reshape(1, -1))
    return out[:, 0]
